# baseline (speedup 1.0000x reference)
import functools
import jax
import jax.numpy as jnp
from jax.experimental import pallas as pl
from jax.experimental.pallas import tpu as pltpu

N = 10000
E = 320000
HID = 128
EDIM = 128
TOWERS = 4
TIN = HID // TOWERS
L = 4
AVG_D_LOG = 3.4965

BE = 2000
NEB = E // BE
BN = 2000
NNB = N // BN


def _init_kernel(hb_ref, p_ref, deg_ref, emb_ref, wp_ref, bp_ref,
                 x_ref, inv_ref, amp_ref, att_ref, has_ref):
    hb = hb_ref[...]
    ids = jax.lax.broadcasted_iota(jnp.int32, hb.shape, 1)
    onehot = (hb.astype(jnp.int32) == ids).astype(jnp.float32)
    x = jnp.dot(onehot, emb_ref[...], preferred_element_type=jnp.float32)
    x = x + jnp.dot(p_ref[...], wp_ref[...], preferred_element_type=jnp.float32)
    x_ref[...] = x + bp_ref[...]

    deg = deg_ref[...]
    degc = jnp.maximum(deg, 1.0)
    log_deg = jnp.log(degc + 1.0)
    inv_ref[...] = 1.0 / degc
    amp_ref[...] = log_deg / AVG_D_LOG
    att_ref[...] = AVG_D_LOG / log_deg
    has_ref[...] = (deg > 0.0).astype(jnp.float32)


def _ab_kernel(x_ref, ws_ref, wd_ref, a_ref, b_ref):
    x = x_ref[...]
    a_ref[...] = jnp.dot(x, ws_ref[...], preferred_element_type=jnp.float32)
    b_ref[...] = jnp.dot(x, wd_ref[...], preferred_element_type=jnp.float32)


def _edge_kernel(src_ref, dst_ref, e_ref, a_ref, b_ref, we_ref,
                 s_ref, q_ref, mx_ref, mn_ref):
    pid = pl.program_id(0)

    @pl.when(pid == 0)
    def _():
        s_ref[...] = jnp.zeros(s_ref.shape, s_ref.dtype)
        q_ref[...] = jnp.zeros(q_ref.shape, q_ref.dtype)
        mx_ref[...] = jnp.zeros(mx_ref.shape, mx_ref.dtype)
        mn_ref[...] = jnp.full(mn_ref.shape, 3.0e38, mn_ref.dtype)

    def flush(d, srun, qrun, mxrun, mnrun):
        s_ref[pl.ds(d, 1), :] = s_ref[pl.ds(d, 1), :] + srun
        q_ref[pl.ds(d, 1), :] = q_ref[pl.ds(d, 1), :] + qrun
        mx_ref[pl.ds(d, 1), :] = jnp.maximum(mx_ref[pl.ds(d, 1), :], mxrun)
        mn_ref[pl.ds(d, 1), :] = jnp.minimum(mn_ref[pl.ds(d, 1), :], mnrun)

    def body(i, carry):
        prev, srun, qrun, mxrun, mnrun = carry
        s = src_ref[0, 0, i]
        d = dst_ref[0, 0, i]
        ee = e_ref[0, 0, i]
        same = d == prev

        @pl.when(jnp.logical_and(jnp.logical_not(same), prev >= 0))
        def _():
            flush(prev, srun, qrun, mxrun, mnrun)

        msg = a_ref[pl.ds(s, 1), :] + b_ref[pl.ds(d, 1), :] + we_ref[pl.ds(ee, 1), :]
        msg = jnp.maximum(msg, 0.0)
        msq = msg * msg
        srun = jnp.where(same, srun + msg, msg)
        qrun = jnp.where(same, qrun + msq, msq)
        mxrun = jnp.where(same, jnp.maximum(mxrun, msg), msg)
        mnrun = jnp.where(same, jnp.minimum(mnrun, msg), msg)
        return d, srun, qrun, mxrun, mnrun

    z = jnp.zeros((1, HID), jnp.float32)
    prev, srun, qrun, mxrun, mnrun = jax.lax.fori_loop(
        0, BE, body, (jnp.int32(-1), z, z, z, z))
    flush(prev, srun, qrun, mxrun, mnrun)


def _post_kernel(x_ref, s_ref, q_ref, mx_ref, mn_ref, inv_ref, amp_ref,
                 att_ref, has_ref, snorm_ref, bmats_ref, bpost_ref,
                 wmix_ref, bmix_ref, hpre_ref, psum_ref, psq_ref):
    pid = pl.program_id(0)
    x = x_ref[...]
    inv = inv_ref[...]
    amp = amp_ref[...]
    att = att_ref[...]
    has = has_ref[...]

    mean = s_ref[...] * inv
    sq = q_ref[...] * inv
    std = jnp.sqrt(jnp.maximum(sq - mean * mean, 0.0) + 1e-5)
    mx = mx_ref[...] * has
    mn = mn_ref[...] * has

    def mm(v, k):
        return jnp.dot(v, bmats_ref[k, :, :], preferred_element_type=jnp.float32)

    hcat = mm(x, 0) + bpost_ref[...]
    aggs = (mean, mx, mn, std)
    for i in range(4):
        a = aggs[i]
        k = 1 + 3 * i
        hcat = hcat + mm(a, k) + amp * mm(a, k + 1) + att * mm(a, k + 2)

    y = jnp.dot(hcat, wmix_ref[...], preferred_element_type=jnp.float32) + bmix_ref[...]
    hmix = jnp.where(y >= 0.0, y, 0.01 * y)
    hpre = hmix * snorm_ref[...]
    hpre_ref[...] = hpre

    @pl.when(pid == 0)
    def _():
        psum_ref[...] = jnp.zeros(psum_ref.shape, psum_ref.dtype)
        psq_ref[...] = jnp.zeros(psq_ref.shape, psq_ref.dtype)

    psum_ref[...] = psum_ref[...] + jnp.sum(hpre, axis=0, keepdims=True)
    psq_ref[...] = psq_ref[...] + jnp.sum(hpre * hpre, axis=0, keepdims=True)


def _bn_kernel(hpre_ref, x_ref, psum_ref, psq_ref, g_ref, bt_ref, o_ref):
    mu = psum_ref[...] / N
    var = psq_ref[...] / N - mu * mu
    hn = (hpre_ref[...] - mu) * jax.lax.rsqrt(var + 1e-5) * g_ref[...] + bt_ref[...]
    o_ref[...] = x_ref[...] + jnp.maximum(hn, 0.0)


def _readout_kernel(x_ref, w1_ref, b1_ref, w2_ref, b2_ref, w3_ref, b3_ref,
                    o_ref, acc_ref):
    pid = pl.program_id(0)

    @pl.when(pid == 0)
    def _():
        acc_ref[...] = jnp.zeros(acc_ref.shape, acc_ref.dtype)

    acc_ref[pl.ds(0, 1), :] = acc_ref[pl.ds(0, 1), :] + jnp.sum(
        x_ref[...], axis=0, keepdims=True)

    @pl.when(pid == NNB - 1)
    def _():
        hg = acc_ref[pl.ds(0, 1), :] / N
        r = jnp.maximum(jnp.dot(hg, w1_ref[...],
                                preferred_element_type=jnp.float32) + b1_ref[...], 0.0)
        r = jnp.maximum(jnp.dot(r, w2_ref[...],
                                preferred_element_type=jnp.float32) + b2_ref[...], 0.0)
        o_ref[...] = jnp.dot(r, w3_ref[...],
                             preferred_element_type=jnp.float32) + b3_ref[...]


def _full(shape):
    return pl.BlockSpec(shape, lambda i: (0,) * len(shape))


def _rows(shape):
    return pl.BlockSpec(shape, lambda i: (i,) + (0,) * (len(shape) - 1))


def _smem_chunk():
    return pl.BlockSpec((1, 1, BE), lambda i: (i, 0, 0), memory_space=pltpu.SMEM)


def kernel(edge_index, h, p, e, snorm_n, hodge_emb, emb_h, Wp, bp, emb_e,
           W_pre, b_pre, W_post, b_post, W_mix, b_mix, bn_gamma, bn_beta,
           W_r1, b_r1, W_r2, b_r2, W_r3, b_r3):
    del hodge_emb
    f32 = jnp.float32
    dst0 = edge_index[1].astype(jnp.int32)
    perm = jnp.argsort(dst0)
    dst_s = dst0[perm]
    src = edge_index[0].astype(jnp.int32)[perm].reshape(NEB, 1, BE)
    dst = dst_s.reshape(NEB, 1, BE)
    ee = e.astype(jnp.int32)[perm].reshape(NEB, 1, BE)
    row_start = jnp.searchsorted(dst_s, jnp.arange(N + 1, dtype=jnp.int32))
    deg_b = jnp.broadcast_to(
        (row_start[1:] - row_start[:N]).astype(f32)[:, None], (N, HID))

    def blockdiag(Wl):
        out = jnp.zeros((HID, HID), f32)
        for t in range(TOWERS):
            out = jax.lax.dynamic_update_slice(out, Wl[t], (t * TIN, t * TIN))
        return out

    hb = jnp.broadcast_to(h.astype(f32)[:, None], (N, 32))
    emb_pad = jnp.zeros((32, HID), f32).at[:28].set(emb_h)
    bp2 = bp.reshape(1, HID)
    snorm_b = jnp.broadcast_to(snorm_n, (N, HID))

    Wsrc, Wdst, Wet, Bm, bpost2, Wmix2, bmix2, g2, bt2 = [], [], [], [], [], [], [], [], []
    for l in range(L):
        Wsrc.append(blockdiag(W_pre[l, :, :TIN, :]))
        Wdst.append(blockdiag(W_pre[l, :, TIN:2 * TIN, :]))
        we = jnp.concatenate([emb_e @ W_pre[l, t, 2 * TIN:, :] + b_pre[l, t]
                              for t in range(TOWERS)], axis=1)
        Wet.append(jnp.zeros((8, HID), f32).at[:4].set(we))
        Bm.append(jnp.stack([blockdiag(W_post[l, :, k * TIN:(k + 1) * TIN, :])
                             for k in range(13)]))
        bpost2.append(b_post[l].reshape(1, HID))
        Wmix2.append(W_mix[l])
        bmix2.append(b_mix[l].reshape(1, HID))
        g2.append(bn_gamma[l].reshape(1, HID))
        bt2.append(bn_beta[l].reshape(1, HID))

    x, inv, amp, att, has = pl.pallas_call(
        _init_kernel,
        grid=(NNB,),
        in_specs=[_rows((BN, 32)), _rows((BN, 8)), _rows((BN, HID)),
                  _full((32, HID)), _full((8, HID)), _full((1, HID))],
        out_specs=[_rows((BN, HID))] * 5,
        out_shape=[jax.ShapeDtypeStruct((N, HID), f32)] * 5,
    )(hb, p, deg_b, emb_pad, Wp, bp2)

    for l in range(L):
        a, b = pl.pallas_call(
            _ab_kernel,
            grid=(NNB,),
            in_specs=[_rows((BN, HID)), _full((HID, HID)), _full((HID, HID))],
            out_specs=[_rows((BN, HID))] * 2,
            out_shape=[jax.ShapeDtypeStruct((N, HID), f32)] * 2,
        )(x, Wsrc[l], Wdst[l])

        S, Q, MX, MN = pl.pallas_call(
            _edge_kernel,
            grid=(NEB,),
            in_specs=[_smem_chunk(), _smem_chunk(), _smem_chunk(),
                      _full((N, HID)), _full((N, HID)), _full((8, HID))],
            out_specs=[_full((N, HID))] * 4,
            out_shape=[jax.ShapeDtypeStruct((N, HID), f32)] * 4,
        )(src, dst, ee, a, b, Wet[l])

        hpre, psum, psq = pl.pallas_call(
            _post_kernel,
            grid=(NNB,),
            in_specs=[_rows((BN, HID))] * 10 + [_full((13, HID, HID)),
                      _full((1, HID)), _full((HID, HID)), _full((1, HID))],
            out_specs=[_rows((BN, HID)), _full((1, HID)), _full((1, HID))],
            out_shape=[jax.ShapeDtypeStruct((N, HID), f32),
                       jax.ShapeDtypeStruct((1, HID), f32),
                       jax.ShapeDtypeStruct((1, HID), f32)],
        )(x, S, Q, MX, MN, inv, amp, att, has, snorm_b,
          Bm[l], bpost2[l], Wmix2[l], bmix2[l])

        x = pl.pallas_call(
            _bn_kernel,
            grid=(NNB,),
            in_specs=[_rows((BN, HID)), _rows((BN, HID)), _full((1, HID)),
                      _full((1, HID)), _full((1, HID)), _full((1, HID))],
            out_specs=_rows((BN, HID)),
            out_shape=jax.ShapeDtypeStruct((N, HID), f32),
        )(hpre, x, psum, psq, g2[l], bt2[l])

    out = pl.pallas_call(
        _readout_kernel,
        grid=(NNB,),
        in_specs=[_rows((BN, HID)), _full((HID, 64)), _full((1, 64)),
                  _full((64, 32)), _full((1, 32)), _full((32, 1)), _full((1, 1))],
        out_specs=_full((1, 1)),
        out_shape=jax.ShapeDtypeStruct((1, 1), f32),
        scratch_shapes=[pltpu.VMEM((8, HID), f32)],
    )(x, W_r1, b_r1.reshape(1, 64), W_r2, b_r2.reshape(1, 32),
      W_r3, b_r3.reshape(1, 1))
    return out.reshape(1)


# speedup vs baseline: 1.5097x; 1.5097x over previous
import jax
import jax.numpy as jnp
from jax.experimental import pallas as pl
from jax.experimental.pallas import tpu as pltpu

N = 10000
E = 320000
HID = 128
EDIM = 128
TOWERS = 4
TIN = HID // TOWERS
L = 4
AVG_D_LOG = 3.4965

BE = 2000
NEB = E // BE
BN = 2000
NNB = N // BN
H2 = 2 * HID


def _deg_kernel(dst_ref, deg_ref):
    pid = pl.program_id(0)

    @pl.when(pid == 0)
    def _():
        deg_ref[...] = jnp.zeros(deg_ref.shape, deg_ref.dtype)

    def body(i, _):
        d = dst_ref[0, 0, i]
        deg_ref[pl.ds(d, 1), :] = deg_ref[pl.ds(d, 1), :] + 1.0
        return 0

    jax.lax.fori_loop(0, BE, body, 0)


def _init_kernel(hb_ref, p_ref, deg_ref, emb_ref, wp_ref, bp_ref,
                 x_ref, inv_ref, amp_ref, att_ref, has_ref):
    hb = hb_ref[...]
    ids = jax.lax.broadcasted_iota(jnp.int32, hb.shape, 1)
    onehot = (hb.astype(jnp.int32) == ids).astype(jnp.float32)
    x = jnp.dot(onehot, emb_ref[...], preferred_element_type=jnp.float32)
    x = x + jnp.dot(p_ref[...], wp_ref[...], preferred_element_type=jnp.float32)
    x_ref[...] = x + bp_ref[...]

    deg = deg_ref[...]
    degc = jnp.maximum(deg, 1.0)
    log_deg = jnp.log(degc + 1.0)
    inv_ref[...] = 1.0 / degc
    amp_ref[...] = log_deg / AVG_D_LOG
    att_ref[...] = AVG_D_LOG / log_deg
    has_ref[...] = (deg > 0.0).astype(jnp.float32)


def _ab_kernel(x_ref, ws_ref, wd_ref, a_ref, b_ref):
    x = x_ref[...]
    a_ref[...] = jnp.dot(x, ws_ref[...], preferred_element_type=jnp.float32)
    b_ref[...] = jnp.dot(x, wd_ref[...], preferred_element_type=jnp.float32)


def _edge_kernel(src_ref, dst_ref, e_ref, a_ref, b_ref, we_ref,
                 sq_ref, mm_ref):
    pid = pl.program_id(0)

    @pl.when(pid == 0)
    def _():
        sq_ref[...] = jnp.zeros(sq_ref.shape, sq_ref.dtype)
        mm_ref[...] = jnp.full(mm_ref.shape, -3.0e38, mm_ref.dtype)

    def body(i, _):
        s = src_ref[0, 0, i]
        d = dst_ref[0, 0, i]
        ee = e_ref[0, 0, i]
        msg = a_ref[pl.ds(s, 1), :] + b_ref[pl.ds(d, 1), :] + we_ref[pl.ds(ee, 1), :]
        msg = jnp.maximum(msg, 0.0)
        sqcat = jnp.concatenate([msg, msg * msg], axis=1)
        mmcat = jnp.concatenate([msg, -msg], axis=1)
        sq_ref[pl.ds(d, 1), :] = sq_ref[pl.ds(d, 1), :] + sqcat
        mm_ref[pl.ds(d, 1), :] = jnp.maximum(mm_ref[pl.ds(d, 1), :], mmcat)
        return 0

    jax.lax.fori_loop(0, BE, body, 0)


def _post_kernel(x_ref, sq_ref, mm_ref, inv_ref, amp_ref,
                 att_ref, has_ref, snorm_ref, bmats_ref, bpost_ref,
                 wmix_ref, bmix_ref, hpre_ref, psum_ref, psq_ref):
    pid = pl.program_id(0)
    x = x_ref[...]
    inv = inv_ref[...]
    amp = amp_ref[...]
    att = att_ref[...]
    has = has_ref[...]

    sqc = sq_ref[...]
    mmc = mm_ref[...]
    mean = sqc[:, :HID] * inv
    sq = sqc[:, HID:] * inv
    std = jnp.sqrt(jnp.maximum(sq - mean * mean, 0.0) + 1e-5)
    mx = mmc[:, :HID] * has
    mn = -mmc[:, HID:] * has

    def mm(v, k):
        return jnp.dot(v, bmats_ref[k, :, :], preferred_element_type=jnp.float32)

    hcat = mm(x, 0) + bpost_ref[...]
    aggs = (mean, mx, mn, std)
    for i in range(4):
        a = aggs[i]
        k = 1 + 3 * i
        hcat = hcat + mm(a, k) + amp * mm(a, k + 1) + att * mm(a, k + 2)

    y = jnp.dot(hcat, wmix_ref[...], preferred_element_type=jnp.float32) + bmix_ref[...]
    hmix = jnp.where(y >= 0.0, y, 0.01 * y)
    hpre = hmix * snorm_ref[...]
    hpre_ref[...] = hpre

    @pl.when(pid == 0)
    def _():
        psum_ref[...] = jnp.zeros(psum_ref.shape, psum_ref.dtype)
        psq_ref[...] = jnp.zeros(psq_ref.shape, psq_ref.dtype)

    psum_ref[...] = psum_ref[...] + jnp.sum(hpre, axis=0, keepdims=True)
    psq_ref[...] = psq_ref[...] + jnp.sum(hpre * hpre, axis=0, keepdims=True)


def _bn_kernel(hpre_ref, x_ref, psum_ref, psq_ref, g_ref, bt_ref, o_ref):
    mu = psum_ref[...] / N
    var = psq_ref[...] / N - mu * mu
    hn = (hpre_ref[...] - mu) * jax.lax.rsqrt(var + 1e-5) * g_ref[...] + bt_ref[...]
    o_ref[...] = x_ref[...] + jnp.maximum(hn, 0.0)


def _readout_kernel(x_ref, w1_ref, b1_ref, w2_ref, b2_ref, w3_ref, b3_ref,
                    o_ref, acc_ref):
    pid = pl.program_id(0)

    @pl.when(pid == 0)
    def _():
        acc_ref[...] = jnp.zeros(acc_ref.shape, acc_ref.dtype)

    acc_ref[pl.ds(0, 1), :] = acc_ref[pl.ds(0, 1), :] + jnp.sum(
        x_ref[...], axis=0, keepdims=True)

    @pl.when(pid == NNB - 1)
    def _():
        hg = acc_ref[pl.ds(0, 1), :] / N
        r = jnp.maximum(jnp.dot(hg, w1_ref[...],
                                preferred_element_type=jnp.float32) + b1_ref[...], 0.0)
        r = jnp.maximum(jnp.dot(r, w2_ref[...],
                                preferred_element_type=jnp.float32) + b2_ref[...], 0.0)
        o_ref[...] = jnp.dot(r, w3_ref[...],
                             preferred_element_type=jnp.float32) + b3_ref[...]


def _full(shape):
    return pl.BlockSpec(shape, lambda i: (0,) * len(shape))


def _rows(shape):
    return pl.BlockSpec(shape, lambda i: (i,) + (0,) * (len(shape) - 1))


def _smem_chunk():
    return pl.BlockSpec((1, 1, BE), lambda i: (i, 0, 0), memory_space=pltpu.SMEM)


def kernel(edge_index, h, p, e, snorm_n, hodge_emb, emb_h, Wp, bp, emb_e,
           W_pre, b_pre, W_post, b_post, W_mix, b_mix, bn_gamma, bn_beta,
           W_r1, b_r1, W_r2, b_r2, W_r3, b_r3):
    del hodge_emb
    f32 = jnp.float32
    src = edge_index[0].astype(jnp.int32).reshape(NEB, 1, BE)
    dst = edge_index[1].astype(jnp.int32).reshape(NEB, 1, BE)
    ee = e.astype(jnp.int32).reshape(NEB, 1, BE)

    def blockdiag(Wl):
        out = jnp.zeros((HID, HID), f32)
        for t in range(TOWERS):
            out = jax.lax.dynamic_update_slice(out, Wl[t], (t * TIN, t * TIN))
        return out

    hb = jnp.broadcast_to(h.astype(f32)[:, None], (N, 32))
    emb_pad = jnp.zeros((32, HID), f32).at[:28].set(emb_h)
    bp2 = bp.reshape(1, HID)
    snorm_b = jnp.broadcast_to(snorm_n, (N, HID))

    Wsrc, Wdst, Wet, Bm, bpost2, Wmix2, bmix2, g2, bt2 = [], [], [], [], [], [], [], [], []
    for l in range(L):
        Wsrc.append(blockdiag(W_pre[l, :, :TIN, :]))
        Wdst.append(blockdiag(W_pre[l, :, TIN:2 * TIN, :]))
        we = jnp.concatenate([emb_e @ W_pre[l, t, 2 * TIN:, :] + b_pre[l, t]
                              for t in range(TOWERS)], axis=1)
        Wet.append(jnp.zeros((8, HID), f32).at[:4].set(we))
        Bm.append(jnp.stack([blockdiag(W_post[l, :, k * TIN:(k + 1) * TIN, :])
                             for k in range(13)]))
        bpost2.append(b_post[l].reshape(1, HID))
        Wmix2.append(W_mix[l])
        bmix2.append(b_mix[l].reshape(1, HID))
        g2.append(bn_gamma[l].reshape(1, HID))
        bt2.append(bn_beta[l].reshape(1, HID))

    deg = pl.pallas_call(
        _deg_kernel,
        grid=(NEB,),
        in_specs=[_smem_chunk()],
        out_specs=_full((N, HID)),
        out_shape=jax.ShapeDtypeStruct((N, HID), f32),
    )(dst)

    x, inv, amp, att, has = pl.pallas_call(
        _init_kernel,
        grid=(NNB,),
        in_specs=[_rows((BN, 32)), _rows((BN, 8)), _rows((BN, HID)),
                  _full((32, HID)), _full((8, HID)), _full((1, HID))],
        out_specs=[_rows((BN, HID))] * 5,
        out_shape=[jax.ShapeDtypeStruct((N, HID), f32)] * 5,
    )(hb, p, deg, emb_pad, Wp, bp2)

    for l in range(L):
        a, b = pl.pallas_call(
            _ab_kernel,
            grid=(NNB,),
            in_specs=[_rows((BN, HID)), _full((HID, HID)), _full((HID, HID))],
            out_specs=[_rows((BN, HID))] * 2,
            out_shape=[jax.ShapeDtypeStruct((N, HID), f32)] * 2,
        )(x, Wsrc[l], Wdst[l])

        SQ, MM = pl.pallas_call(
            _edge_kernel,
            grid=(NEB,),
            in_specs=[_smem_chunk(), _smem_chunk(), _smem_chunk(),
                      _full((N, HID)), _full((N, HID)), _full((8, HID))],
            out_specs=[_full((N, H2))] * 2,
            out_shape=[jax.ShapeDtypeStruct((N, H2), f32)] * 2,
        )(src, dst, ee, a, b, Wet[l])

        hpre, psum, psq = pl.pallas_call(
            _post_kernel,
            grid=(NNB,),
            in_specs=[_rows((BN, HID)), _rows((BN, H2)), _rows((BN, H2))] +
                     [_rows((BN, HID))] * 5 + [_full((13, HID, HID)),
                      _full((1, HID)), _full((HID, HID)), _full((1, HID))],
            out_specs=[_rows((BN, HID)), _full((1, HID)), _full((1, HID))],
            out_shape=[jax.ShapeDtypeStruct((N, HID), f32),
                       jax.ShapeDtypeStruct((1, HID), f32),
                       jax.ShapeDtypeStruct((1, HID), f32)],
        )(x, SQ, MM, inv, amp, att, has, snorm_b,
          Bm[l], bpost2[l], Wmix2[l], bmix2[l])

        x = pl.pallas_call(
            _bn_kernel,
            grid=(NNB,),
            in_specs=[_rows((BN, HID)), _rows((BN, HID)), _full((1, HID)),
                      _full((1, HID)), _full((1, HID)), _full((1, HID))],
            out_specs=_rows((BN, HID)),
            out_shape=jax.ShapeDtypeStruct((N, HID), f32),
        )(hpre, x, psum, psq, g2[l], bt2[l])

    out = pl.pallas_call(
        _readout_kernel,
        grid=(NNB,),
        in_specs=[_rows((BN, HID)), _full((HID, 64)), _full((1, 64)),
                  _full((64, 32)), _full((1, 32)), _full((32, 1)), _full((1, 1))],
        out_specs=_full((1, 1)),
        out_shape=jax.ShapeDtypeStruct((1, 1), f32),
        scratch_shapes=[pltpu.VMEM((8, HID), f32)],
    )(x, W_r1, b_r1.reshape(1, 64), W_r2, b_r2.reshape(1, 32),
      W_r3, b_r3.reshape(1, 1))
    return out.reshape(1)


# speedup vs baseline: 3.5039x; 2.3209x over previous
import jax
import jax.numpy as jnp
from jax.experimental import pallas as pl
from jax.experimental.pallas import tpu as pltpu

N = 10000
E = 320000
HID = 128
EDIM = 128
TOWERS = 4
TIN = HID // TOWERS
L = 4
AVG_D_LOG = 3.4965

BE = 2000
NEB = E // BE
BN = 2000
NNB = N // BN
H2 = 2 * HID


def _deg_kernel(dst_ref, deg_ref):
    pid = pl.program_id(0)

    @pl.when(pid == 0)
    def _():
        deg_ref[...] = jnp.zeros(deg_ref.shape, deg_ref.dtype)

    def body(i, _):
        d = dst_ref[0, 0, i]
        deg_ref[pl.ds(d, 1), :] = deg_ref[pl.ds(d, 1), :] + 1.0
        return 0

    jax.lax.fori_loop(0, BE, body, 0, unroll=8)


def _init_kernel(hb_ref, p_ref, deg_ref, emb_ref, wp_ref, bp_ref,
                 x_ref, inv_ref, amp_ref, att_ref, has_ref):
    hb = hb_ref[...]
    ids = jax.lax.broadcasted_iota(jnp.int32, hb.shape, 1)
    onehot = (hb.astype(jnp.int32) == ids).astype(jnp.float32)
    x = jnp.dot(onehot, emb_ref[...], preferred_element_type=jnp.float32)
    x = x + jnp.dot(p_ref[...], wp_ref[...], preferred_element_type=jnp.float32)
    x_ref[...] = x + bp_ref[...]

    deg = deg_ref[...]
    degc = jnp.maximum(deg, 1.0)
    log_deg = jnp.log(degc + 1.0)
    inv_ref[...] = 1.0 / degc
    amp_ref[...] = log_deg / AVG_D_LOG
    att_ref[...] = AVG_D_LOG / log_deg
    has_ref[...] = (deg > 0.0).astype(jnp.float32)


def _ab_kernel(x_ref, ws_ref, wd_ref, a_ref, b_ref):
    x = x_ref[...]
    a_ref[...] = jnp.dot(x, ws_ref[...], preferred_element_type=jnp.float32)
    b_ref[...] = jnp.dot(x, wd_ref[...], preferred_element_type=jnp.float32)


def _edge_kernel(src_ref, dst_ref, e_ref, a_ref, b_ref, we_ref,
                 s_ref, q_ref, mx_ref, mn_ref):
    pid = pl.program_id(0)

    @pl.when(pid == 0)
    def _():
        s_ref[...] = jnp.zeros(s_ref.shape, s_ref.dtype)
        q_ref[...] = jnp.zeros(q_ref.shape, q_ref.dtype)
        mx_ref[...] = jnp.zeros(mx_ref.shape, mx_ref.dtype)
        mn_ref[...] = jnp.full(mn_ref.shape, 3.0e38, mn_ref.dtype)

    def body(i, _):
        s = src_ref[0, 0, i]
        d = dst_ref[0, 0, i]
        ee = e_ref[0, 0, i]
        msg = a_ref[pl.ds(s, 1), :] + b_ref[pl.ds(d, 1), :] + we_ref[pl.ds(ee, 1), :]
        msg = jnp.maximum(msg, 0.0)
        s_ref[pl.ds(d, 1), :] = s_ref[pl.ds(d, 1), :] + msg
        q_ref[pl.ds(d, 1), :] = q_ref[pl.ds(d, 1), :] + msg * msg
        mx_ref[pl.ds(d, 1), :] = jnp.maximum(mx_ref[pl.ds(d, 1), :], msg)
        mn_ref[pl.ds(d, 1), :] = jnp.minimum(mn_ref[pl.ds(d, 1), :], msg)
        return 0

    jax.lax.fori_loop(0, BE, body, 0, unroll=8)


def _post_kernel(x_ref, s_ref, q_ref, mx_ref, mn_ref, inv_ref, amp_ref,
                 att_ref, has_ref, snorm_ref, bmats_ref, bpost_ref,
                 wmix_ref, bmix_ref, hpre_ref, psum_ref, psq_ref):
    pid = pl.program_id(0)
    x = x_ref[...]
    inv = inv_ref[...]
    amp = amp_ref[...]
    att = att_ref[...]
    has = has_ref[...]

    mean = s_ref[...] * inv
    sq = q_ref[...] * inv
    std = jnp.sqrt(jnp.maximum(sq - mean * mean, 0.0) + 1e-5)
    mx = mx_ref[...] * has
    mn = mn_ref[...] * has

    def mm(v, k):
        return jnp.dot(v, bmats_ref[k, :, :], preferred_element_type=jnp.float32)

    hcat = mm(x, 0) + bpost_ref[...]
    aggs = (mean, mx, mn, std)
    for i in range(4):
        a = aggs[i]
        k = 1 + 3 * i
        hcat = hcat + mm(a, k) + amp * mm(a, k + 1) + att * mm(a, k + 2)

    y = jnp.dot(hcat, wmix_ref[...], preferred_element_type=jnp.float32) + bmix_ref[...]
    hmix = jnp.where(y >= 0.0, y, 0.01 * y)
    hpre = hmix * snorm_ref[...]
    hpre_ref[...] = hpre

    @pl.when(pid == 0)
    def _():
        psum_ref[...] = jnp.zeros(psum_ref.shape, psum_ref.dtype)
        psq_ref[...] = jnp.zeros(psq_ref.shape, psq_ref.dtype)

    psum_ref[...] = psum_ref[...] + jnp.sum(hpre, axis=0, keepdims=True)
    psq_ref[...] = psq_ref[...] + jnp.sum(hpre * hpre, axis=0, keepdims=True)


def _bn_kernel(hpre_ref, x_ref, psum_ref, psq_ref, g_ref, bt_ref, o_ref):
    mu = psum_ref[...] / N
    var = psq_ref[...] / N - mu * mu
    hn = (hpre_ref[...] - mu) * jax.lax.rsqrt(var + 1e-5) * g_ref[...] + bt_ref[...]
    o_ref[...] = x_ref[...] + jnp.maximum(hn, 0.0)


def _readout_kernel(x_ref, w1_ref, b1_ref, w2_ref, b2_ref, w3_ref, b3_ref,
                    o_ref, acc_ref):
    pid = pl.program_id(0)

    @pl.when(pid == 0)
    def _():
        acc_ref[...] = jnp.zeros(acc_ref.shape, acc_ref.dtype)

    acc_ref[pl.ds(0, 1), :] = acc_ref[pl.ds(0, 1), :] + jnp.sum(
        x_ref[...], axis=0, keepdims=True)

    @pl.when(pid == NNB - 1)
    def _():
        hg = acc_ref[pl.ds(0, 1), :] / N
        r = jnp.maximum(jnp.dot(hg, w1_ref[...],
                                preferred_element_type=jnp.float32) + b1_ref[...], 0.0)
        r = jnp.maximum(jnp.dot(r, w2_ref[...],
                                preferred_element_type=jnp.float32) + b2_ref[...], 0.0)
        o_ref[...] = jnp.dot(r, w3_ref[...],
                             preferred_element_type=jnp.float32) + b3_ref[...]


def _full(shape):
    return pl.BlockSpec(shape, lambda i: (0,) * len(shape))


def _rows(shape):
    return pl.BlockSpec(shape, lambda i: (i,) + (0,) * (len(shape) - 1))


def _smem_chunk():
    return pl.BlockSpec((1, 1, BE), lambda i: (i, 0, 0), memory_space=pltpu.SMEM)


def kernel(edge_index, h, p, e, snorm_n, hodge_emb, emb_h, Wp, bp, emb_e,
           W_pre, b_pre, W_post, b_post, W_mix, b_mix, bn_gamma, bn_beta,
           W_r1, b_r1, W_r2, b_r2, W_r3, b_r3):
    del hodge_emb
    f32 = jnp.float32
    src = edge_index[0].astype(jnp.int32).reshape(NEB, 1, BE)
    dst = edge_index[1].astype(jnp.int32).reshape(NEB, 1, BE)
    ee = e.astype(jnp.int32).reshape(NEB, 1, BE)

    def blockdiag(Wl):
        out = jnp.zeros((HID, HID), f32)
        for t in range(TOWERS):
            out = jax.lax.dynamic_update_slice(out, Wl[t], (t * TIN, t * TIN))
        return out

    hb = jnp.broadcast_to(h.astype(f32)[:, None], (N, 32))
    emb_pad = jnp.zeros((32, HID), f32).at[:28].set(emb_h)
    bp2 = bp.reshape(1, HID)
    snorm_b = jnp.broadcast_to(snorm_n, (N, HID))

    Wsrc, Wdst, Wet, Bm, bpost2, Wmix2, bmix2, g2, bt2 = [], [], [], [], [], [], [], [], []
    for l in range(L):
        Wsrc.append(blockdiag(W_pre[l, :, :TIN, :]))
        Wdst.append(blockdiag(W_pre[l, :, TIN:2 * TIN, :]))
        we = jnp.concatenate([emb_e @ W_pre[l, t, 2 * TIN:, :] + b_pre[l, t]
                              for t in range(TOWERS)], axis=1)
        Wet.append(jnp.zeros((8, HID), f32).at[:4].set(we))
        Bm.append(jnp.stack([blockdiag(W_post[l, :, k * TIN:(k + 1) * TIN, :])
                             for k in range(13)]))
        bpost2.append(b_post[l].reshape(1, HID))
        Wmix2.append(W_mix[l])
        bmix2.append(b_mix[l].reshape(1, HID))
        g2.append(bn_gamma[l].reshape(1, HID))
        bt2.append(bn_beta[l].reshape(1, HID))

    deg = pl.pallas_call(
        _deg_kernel,
        grid=(NEB,),
        in_specs=[_smem_chunk()],
        out_specs=_full((N, HID)),
        out_shape=jax.ShapeDtypeStruct((N, HID), f32),
    )(dst)

    x, inv, amp, att, has = pl.pallas_call(
        _init_kernel,
        grid=(NNB,),
        in_specs=[_rows((BN, 32)), _rows((BN, 8)), _rows((BN, HID)),
                  _full((32, HID)), _full((8, HID)), _full((1, HID))],
        out_specs=[_rows((BN, HID))] * 5,
        out_shape=[jax.ShapeDtypeStruct((N, HID), f32)] * 5,
    )(hb, p, deg, emb_pad, Wp, bp2)

    for l in range(L):
        a, b = pl.pallas_call(
            _ab_kernel,
            grid=(NNB,),
            in_specs=[_rows((BN, HID)), _full((HID, HID)), _full((HID, HID))],
            out_specs=[_rows((BN, HID))] * 2,
            out_shape=[jax.ShapeDtypeStruct((N, HID), f32)] * 2,
        )(x, Wsrc[l], Wdst[l])

        S, Q, MX, MN = pl.pallas_call(
            _edge_kernel,
            grid=(NEB,),
            in_specs=[_smem_chunk(), _smem_chunk(), _smem_chunk(),
                      _full((N, HID)), _full((N, HID)), _full((8, HID))],
            out_specs=[_full((N, HID))] * 4,
            out_shape=[jax.ShapeDtypeStruct((N, HID), f32)] * 4,
        )(src, dst, ee, a, b, Wet[l])

        hpre, psum, psq = pl.pallas_call(
            _post_kernel,
            grid=(NNB,),
            in_specs=[_rows((BN, HID))] * 10 + [_full((13, HID, HID)),
                      _full((1, HID)), _full((HID, HID)), _full((1, HID))],
            out_specs=[_rows((BN, HID)), _full((1, HID)), _full((1, HID))],
            out_shape=[jax.ShapeDtypeStruct((N, HID), f32),
                       jax.ShapeDtypeStruct((1, HID), f32),
                       jax.ShapeDtypeStruct((1, HID), f32)],
        )(x, S, Q, MX, MN, inv, amp, att, has, snorm_b,
          Bm[l], bpost2[l], Wmix2[l], bmix2[l])

        x = pl.pallas_call(
            _bn_kernel,
            grid=(NNB,),
            in_specs=[_rows((BN, HID)), _rows((BN, HID)), _full((1, HID)),
                      _full((1, HID)), _full((1, HID)), _full((1, HID))],
            out_specs=_rows((BN, HID)),
            out_shape=jax.ShapeDtypeStruct((N, HID), f32),
        )(hpre, x, psum, psq, g2[l], bt2[l])

    out = pl.pallas_call(
        _readout_kernel,
        grid=(NNB,),
        in_specs=[_rows((BN, HID)), _full((HID, 64)), _full((1, 64)),
                  _full((64, 32)), _full((1, 32)), _full((32, 1)), _full((1, 1))],
        out_specs=_full((1, 1)),
        out_shape=jax.ShapeDtypeStruct((1, 1), f32),
        scratch_shapes=[pltpu.VMEM((8, HID), f32)],
    )(x, W_r1, b_r1.reshape(1, 64), W_r2, b_r2.reshape(1, 32),
      W_r3, b_r3.reshape(1, 1))
    return out.reshape(1)
